# baseline (speedup 1.0000x reference)
import functools

import jax
import jax.numpy as jnp
from jax import lax
from jax.experimental import pallas as pl
from jax.experimental.pallas import tpu as pltpu
from jax.experimental.pallas import tpu_sc as plsc

B = 2048
K = 1024
D = 128

NC = 2
NS = 16
L = 16
NW = NC * NS
BPW = B // NW


def _tc_top2_body(w_ref, x_ref, i1_ref, i2_ref):
    w = w_ref[...]
    x = x_ref[...]
    wn = jnp.sum(w * w, axis=1, keepdims=True)
    xw = lax.dot_general(
        w, x,
        dimension_numbers=(((1,), (1,)), ((), ())),
        preferred_element_type=jnp.float32,
        precision=lax.Precision.HIGHEST,
    )
    s = 0.5 * wn - xw
    io = lax.broadcasted_iota(jnp.int32, s.shape, 0)
    m1 = jnp.min(s, axis=0)
    i1 = jnp.min(jnp.where(s == m1[None, :], io, K), axis=0)
    s2 = jnp.where(io == i1[None, :], jnp.inf, s)
    m2 = jnp.min(s2, axis=0)
    i2 = jnp.min(jnp.where(s2 == m2[None, :], io, K), axis=0)
    i1_ref[...] = i1
    i2_ref[...] = i2


def _tc_top2(weights, inputs):
    return pl.pallas_call(
        _tc_top2_body,
        out_shape=[
            jax.ShapeDtypeStruct((B,), jnp.int32),
            jax.ShapeDtypeStruct((B,), jnp.int32),
        ],
    )(weights, inputs)


def _sc_refine_body(x_hbm, w_hbm, grid_hbm, i1_hbm, i2_hbm, out_hbm,
                    idx1_v, idx2_v, x_v, w1_v, w2_v, grid_v, out_v,
                    sem_idx, sem):
    wid = lax.axis_index("s") * NC + lax.axis_index("c")
    base = wid * BPW
    a1 = pltpu.async_copy(i1_hbm.at[pl.ds(base, BPW)], idx1_v, sem_idx)
    a2 = pltpu.async_copy(i2_hbm.at[pl.ds(base, BPW)], idx2_v, sem_idx)
    a3 = pltpu.async_copy(grid_hbm, grid_v, sem)
    a4 = pltpu.async_copy(x_hbm.at[pl.ds(base, BPW)], x_v, sem)
    a1.wait()
    a2.wait()
    c1 = pltpu.async_copy(w_hbm.at[idx1_v], w1_v, sem)
    c2 = pltpu.async_copy(w_hbm.at[idx2_v], w2_v, sem)
    a3.wait()
    a4.wait()
    c1.wait()
    c2.wait()

    zero = jnp.zeros((L,), jnp.float32)
    zeros_i = jnp.zeros((L,), jnp.int32)
    ones_i = zeros_i + 1
    for g in range(BPW // L):
        rows = lax.iota(jnp.int32, L) + g * L
        i1v = idx1_v[pl.ds(g * L, L)]
        i2v = idx2_v[pl.ds(g * L, L)]

        def body(t, carry):
            a1_, k1, a2_, k2 = carry
            tv = jnp.full((L,), t, jnp.int32)
            xv = plsc.load_gather(x_v, [rows, tv])
            w1 = plsc.load_gather(w1_v, [rows, tv])
            w2 = plsc.load_gather(w2_v, [rows, tv])
            d1 = xv - w1
            d2 = xv - w2
            y1 = d1 * d1 - k1
            t1 = a1_ + y1
            k1n = (t1 - a1_) - y1
            y2 = d2 * d2 - k2
            t2 = a2_ + y2
            k2n = (t2 - a2_) - y2
            return (t1, k1n, t2, k2n)

        d1, _, d2, _ = lax.fori_loop(0, D, body, (zero, zero, zero, zero),
                                     unroll=16)
        take2 = (d2 < d1) | ((d2 == d1) & (i2v < i1v))
        win = jnp.where(take2, i2v, i1v)
        gx = plsc.load_gather(grid_v, [win, zeros_i])
        gy = plsc.load_gather(grid_v, [win, ones_i])
        plsc.store_scatter(out_v, [rows, zeros_i], gx)
        plsc.store_scatter(out_v, [rows, ones_i], gy)

    pltpu.sync_copy(out_v, out_hbm.at[pl.ds(base, BPW)])


@functools.cache
def _sc_refine():
    return pl.kernel(
        _sc_refine_body,
        mesh=plsc.VectorSubcoreMesh(
            core_axis_name="c", subcore_axis_name="s",
            num_cores=NC, num_subcores=NS,
        ),
        compiler_params=pltpu.CompilerParams(
            needs_layout_passes=False, use_tc_tiling_on_sc=False,
        ),
        out_type=jax.ShapeDtypeStruct((B, 2), jnp.int32),
        scratch_types=[
            pltpu.VMEM((BPW,), jnp.int32),
            pltpu.VMEM((BPW,), jnp.int32),
            pltpu.VMEM((BPW, D), jnp.float32),
            pltpu.VMEM((BPW, D), jnp.float32),
            pltpu.VMEM((BPW, D), jnp.float32),
            pltpu.VMEM((K, 2), jnp.int32),
            pltpu.VMEM((BPW, 2), jnp.int32),
            pltpu.SemaphoreType.DMA,
            pltpu.SemaphoreType.DMA,
        ],
    )


def kernel(inputs, weights, grid):
    i1, i2 = _tc_top2(weights, inputs)
    return _sc_refine()(inputs, weights, grid, i1, i2)


# speedup vs baseline: 1.0681x; 1.0681x over previous
import functools

import jax
import jax.numpy as jnp
from jax import lax
from jax.experimental import pallas as pl
from jax.experimental.pallas import tpu as pltpu
from jax.experimental.pallas import tpu_sc as plsc

B = 2048
K = 1024
D = 128

NC = 2
NS = 16
L = 16
NW = NC * NS
BPW = B // NW


def _tc_top2_body(w_ref, x_ref, i1_ref, i2_ref):
    w = w_ref[...]
    x = x_ref[...]
    wn = jnp.sum(w * w, axis=1, keepdims=True)
    xw = lax.dot_general(
        w, x,
        dimension_numbers=(((1,), (1,)), ((), ())),
        preferred_element_type=jnp.float32,
        precision=lax.Precision.HIGHEST,
    )
    s = 0.5 * wn - xw
    io = lax.broadcasted_iota(jnp.int32, s.shape, 0)
    m1 = jnp.min(s, axis=0)
    i1 = jnp.min(jnp.where(s == m1[None, :], io, K), axis=0)
    s2 = jnp.where(io == i1[None, :], jnp.inf, s)
    m2 = jnp.min(s2, axis=0)
    i2 = jnp.min(jnp.where(s2 == m2[None, :], io, K), axis=0)
    i1_ref[...] = i1
    i2_ref[...] = i2


def _tc_top2(weights, inputs):
    return pl.pallas_call(
        _tc_top2_body,
        out_shape=[
            jax.ShapeDtypeStruct((B,), jnp.int32),
            jax.ShapeDtypeStruct((B,), jnp.int32),
        ],
    )(weights, inputs)


def _sc_refine_body(x_hbm, w_hbm, grid_hbm, i1_hbm, i2_hbm, out_hbm,
                    idx1_v, idx2_v, x_v, w1_v, w2_v, grid_v, out_v,
                    sem_idx, sem):
    wid = lax.axis_index("s") * NC + lax.axis_index("c")
    base = wid * BPW
    a1 = pltpu.async_copy(i1_hbm.at[pl.ds(base, BPW)], idx1_v, sem_idx)
    a2 = pltpu.async_copy(i2_hbm.at[pl.ds(base, BPW)], idx2_v, sem_idx)
    a3 = pltpu.async_copy(grid_hbm, grid_v, sem)
    a4 = pltpu.async_copy(x_hbm.at[pl.ds(base, BPW)], x_v, sem)
    a1.wait()
    a2.wait()
    c1 = pltpu.async_copy(w_hbm.at[idx1_v], w1_v, sem)
    c2 = pltpu.async_copy(w_hbm.at[idx2_v], w2_v, sem)
    a3.wait()
    a4.wait()
    c1.wait()
    c2.wait()

    NACC = 8
    zero = jnp.zeros((L,), jnp.float32)
    zeros_i = jnp.zeros((L,), jnp.int32)
    ones_i = zeros_i + 1
    for g in range(BPW // L):
        rows = lax.iota(jnp.int32, L) + g * L
        i1v = idx1_v[pl.ds(g * L, L)]
        i2v = idx2_v[pl.ds(g * L, L)]

        def body(step, carry):
            acc1 = list(carry[:NACC])
            acc2 = list(carry[NACC:])
            t0 = step * NACC
            for u in range(NACC):
                tv = jnp.full((L,), t0 + u, jnp.int32)
                xv = plsc.load_gather(x_v, [rows, tv])
                w1 = plsc.load_gather(w1_v, [rows, tv])
                w2 = plsc.load_gather(w2_v, [rows, tv])
                d1 = xv - w1
                d2 = xv - w2
                acc1[u] = acc1[u] + d1 * d1
                acc2[u] = acc2[u] + d2 * d2
            return tuple(acc1) + tuple(acc2)

        accs = lax.fori_loop(0, D // NACC, body, (zero,) * (2 * NACC),
                             unroll=2)
        acc1 = list(accs[:NACC])
        acc2 = list(accs[NACC:])
        step = NACC
        while step > 1:
            step //= 2
            for u in range(step):
                acc1[u] = acc1[u] + acc1[u + step]
                acc2[u] = acc2[u] + acc2[u + step]
        d1 = acc1[0]
        d2 = acc2[0]
        take2 = (d2 < d1) | ((d2 == d1) & (i2v < i1v))
        win = jnp.where(take2, i2v, i1v)
        gx = plsc.load_gather(grid_v, [win, zeros_i])
        gy = plsc.load_gather(grid_v, [win, ones_i])
        plsc.store_scatter(out_v, [rows, zeros_i], gx)
        plsc.store_scatter(out_v, [rows, ones_i], gy)

    pltpu.sync_copy(out_v, out_hbm.at[pl.ds(base, BPW)])


@functools.cache
def _sc_refine():
    return pl.kernel(
        _sc_refine_body,
        mesh=plsc.VectorSubcoreMesh(
            core_axis_name="c", subcore_axis_name="s",
            num_cores=NC, num_subcores=NS,
        ),
        compiler_params=pltpu.CompilerParams(
            needs_layout_passes=False, use_tc_tiling_on_sc=False,
        ),
        out_type=jax.ShapeDtypeStruct((B, 2), jnp.int32),
        scratch_types=[
            pltpu.VMEM((BPW,), jnp.int32),
            pltpu.VMEM((BPW,), jnp.int32),
            pltpu.VMEM((BPW, D), jnp.float32),
            pltpu.VMEM((BPW, D), jnp.float32),
            pltpu.VMEM((BPW, D), jnp.float32),
            pltpu.VMEM((K, 2), jnp.int32),
            pltpu.VMEM((BPW, 2), jnp.int32),
            pltpu.SemaphoreType.DMA,
            pltpu.SemaphoreType.DMA,
        ],
    )


def kernel(inputs, weights, grid):
    i1, i2 = _tc_top2(weights, inputs)
    return _sc_refine()(inputs, weights, grid, i1, i2)


# speedup vs baseline: 1.3670x; 1.2799x over previous
import functools

import jax
import jax.numpy as jnp
from jax import lax
from jax.experimental import pallas as pl
from jax.experimental.pallas import tpu as pltpu
from jax.experimental.pallas import tpu_sc as plsc

B = 2048
K = 1024
D = 128

NC = 2
NS = 16
L = 16
NW = NC * NS
BPW = B // NW


def _tc_top2_body(w_ref, x_ref, i1_ref, i2_ref):
    w = w_ref[...]
    x = x_ref[...]
    wn = jnp.sum(w * w, axis=1, keepdims=True)
    xw = lax.dot_general(
        w, x,
        dimension_numbers=(((1,), (1,)), ((), ())),
        preferred_element_type=jnp.float32,
        precision=lax.Precision.HIGHEST,
    )
    s = 0.5 * wn - xw
    io = lax.broadcasted_iota(jnp.int32, s.shape, 0)
    m1 = jnp.min(s, axis=0)
    i1 = jnp.min(jnp.where(s == m1[None, :], io, K), axis=0)
    s2 = jnp.where(io == i1[None, :], jnp.inf, s)
    m2 = jnp.min(s2, axis=0)
    i2 = jnp.min(jnp.where(s2 == m2[None, :], io, K), axis=0)
    i1_ref[...] = i1
    i2_ref[...] = i2


def _tc_top2(weights, inputs):
    return pl.pallas_call(
        _tc_top2_body,
        out_shape=[
            jax.ShapeDtypeStruct((B,), jnp.int32),
            jax.ShapeDtypeStruct((B,), jnp.int32),
        ],
    )(weights, inputs)


def _sc_refine_body(x_hbm, w_hbm, grid_hbm, i1_hbm, i2_hbm, out_hbm,
                    idx1_v, idx2_v, x_v, w1_v, w2_v, grid_v, out_v,
                    sem_idx, sem):
    wid = lax.axis_index("s") * NC + lax.axis_index("c")
    base = wid * BPW
    a1 = pltpu.async_copy(i1_hbm.at[pl.ds(base, BPW)], idx1_v, sem_idx)
    a2 = pltpu.async_copy(i2_hbm.at[pl.ds(base, BPW)], idx2_v, sem_idx)
    a3 = pltpu.async_copy(grid_hbm, grid_v, sem)
    a4 = pltpu.async_copy(x_hbm.at[pl.ds(base, BPW)], x_v, sem)
    a1.wait()
    a2.wait()
    c1 = pltpu.async_copy(w_hbm.at[idx1_v], w1_v, sem)
    c2 = pltpu.async_copy(w_hbm.at[idx2_v], w2_v, sem)
    a3.wait()
    a4.wait()
    c1.wait()
    c2.wait()

    NACC = 8
    zero = jnp.zeros((L,), jnp.float32)
    zeros_i = jnp.zeros((L,), jnp.int32)
    ones_i = zeros_i + 1
    stag = (9 * lax.iota(jnp.int32, L)) % D
    for g in range(BPW // L):
        rows = lax.iota(jnp.int32, L) + g * L
        i1v = idx1_v[pl.ds(g * L, L)]
        i2v = idx2_v[pl.ds(g * L, L)]

        def body(step, carry):
            acc1 = list(carry[:NACC])
            acc2 = list(carry[NACC:])
            t0 = step * NACC
            for u in range(NACC):
                tv = stag + (t0 + u)
                tv = jnp.where(tv >= D, tv - D, tv)
                xv = plsc.load_gather(x_v, [rows, tv])
                w1 = plsc.load_gather(w1_v, [rows, tv])
                w2 = plsc.load_gather(w2_v, [rows, tv])
                d1 = xv - w1
                d2 = xv - w2
                acc1[u] = acc1[u] + d1 * d1
                acc2[u] = acc2[u] + d2 * d2
            return tuple(acc1) + tuple(acc2)

        accs = lax.fori_loop(0, D // NACC, body, (zero,) * (2 * NACC),
                             unroll=2)
        acc1 = list(accs[:NACC])
        acc2 = list(accs[NACC:])
        step = NACC
        while step > 1:
            step //= 2
            for u in range(step):
                acc1[u] = acc1[u] + acc1[u + step]
                acc2[u] = acc2[u] + acc2[u + step]
        d1 = acc1[0]
        d2 = acc2[0]
        take2 = (d2 < d1) | ((d2 == d1) & (i2v < i1v))
        win = jnp.where(take2, i2v, i1v)
        gx = plsc.load_gather(grid_v, [win, zeros_i])
        gy = plsc.load_gather(grid_v, [win, ones_i])
        plsc.store_scatter(out_v, [rows, zeros_i], gx)
        plsc.store_scatter(out_v, [rows, ones_i], gy)

    pltpu.sync_copy(out_v, out_hbm.at[pl.ds(base, BPW)])


@functools.cache
def _sc_refine():
    return pl.kernel(
        _sc_refine_body,
        mesh=plsc.VectorSubcoreMesh(
            core_axis_name="c", subcore_axis_name="s",
            num_cores=NC, num_subcores=NS,
        ),
        compiler_params=pltpu.CompilerParams(
            needs_layout_passes=False, use_tc_tiling_on_sc=False,
        ),
        out_type=jax.ShapeDtypeStruct((B, 2), jnp.int32),
        scratch_types=[
            pltpu.VMEM((BPW,), jnp.int32),
            pltpu.VMEM((BPW,), jnp.int32),
            pltpu.VMEM((BPW, D), jnp.float32),
            pltpu.VMEM((BPW, D), jnp.float32),
            pltpu.VMEM((BPW, D), jnp.float32),
            pltpu.VMEM((K, 2), jnp.int32),
            pltpu.VMEM((BPW, 2), jnp.int32),
            pltpu.SemaphoreType.DMA,
            pltpu.SemaphoreType.DMA,
        ],
    )


def kernel(inputs, weights, grid):
    i1, i2 = _tc_top2(weights, inputs)
    return _sc_refine()(inputs, weights, grid, i1, i2)


# speedup vs baseline: 1.4085x; 1.0304x over previous
import functools

import jax
import jax.numpy as jnp
from jax import lax
from jax.experimental import pallas as pl
from jax.experimental.pallas import tpu as pltpu
from jax.experimental.pallas import tpu_sc as plsc

B = 2048
K = 1024
D = 128

NC = 2
NS = 16
L = 16
NW = NC * NS
BPW = B // NW


def _tc_top2_body(w_ref, x_ref, i1_ref, i2_ref):
    w = w_ref[...]
    x = x_ref[...]
    wn = jnp.sum(w * w, axis=1, keepdims=True)
    xw = lax.dot_general(
        w, x,
        dimension_numbers=(((1,), (1,)), ((), ())),
        preferred_element_type=jnp.float32,
        precision=lax.Precision.HIGHEST,
    )
    s = 0.5 * wn - xw
    i1 = jnp.argmin(s, axis=0).astype(jnp.int32)
    io = lax.broadcasted_iota(jnp.int32, s.shape, 0)
    s2 = jnp.where(io == i1[None, :], jnp.inf, s)
    i2 = jnp.argmin(s2, axis=0).astype(jnp.int32)
    i1_ref[...] = i1
    i2_ref[...] = i2


def _tc_top2(weights, inputs):
    return pl.pallas_call(
        _tc_top2_body,
        out_shape=[
            jax.ShapeDtypeStruct((B,), jnp.int32),
            jax.ShapeDtypeStruct((B,), jnp.int32),
        ],
    )(weights, inputs)


def _sc_refine_body(x_hbm, w_hbm, grid_hbm, i1_hbm, i2_hbm, out_hbm,
                    idx1_v, idx2_v, x_v, w1_v, w2_v, grid_v, out_v,
                    sem_idx, sem):
    wid = lax.axis_index("s") * NC + lax.axis_index("c")
    base = wid * BPW
    a1 = pltpu.async_copy(i1_hbm.at[pl.ds(base, BPW)], idx1_v, sem_idx)
    a2 = pltpu.async_copy(i2_hbm.at[pl.ds(base, BPW)], idx2_v, sem_idx)
    a3 = pltpu.async_copy(grid_hbm, grid_v, sem)
    a4 = pltpu.async_copy(x_hbm.at[pl.ds(base, BPW)], x_v, sem)
    a1.wait()
    a2.wait()
    c1 = pltpu.async_copy(w_hbm.at[idx1_v], w1_v, sem)
    c2 = pltpu.async_copy(w_hbm.at[idx2_v], w2_v, sem)
    a3.wait()
    a4.wait()
    c1.wait()
    c2.wait()

    NACC = 8
    zero = jnp.zeros((L,), jnp.float32)
    zeros_i = jnp.zeros((L,), jnp.int32)
    ones_i = zeros_i + 1
    stag = (9 * lax.iota(jnp.int32, L)) % D
    for g in range(BPW // L):
        rows = lax.iota(jnp.int32, L) + g * L
        i1v = idx1_v[pl.ds(g * L, L)]
        i2v = idx2_v[pl.ds(g * L, L)]

        def body(step, carry):
            acc1 = list(carry[:NACC])
            acc2 = list(carry[NACC:])
            t0 = step * NACC
            for u in range(NACC):
                tv = stag + (t0 + u)
                tv = jnp.where(tv >= D, tv - D, tv)
                xv = plsc.load_gather(x_v, [rows, tv])
                w1 = plsc.load_gather(w1_v, [rows, tv])
                w2 = plsc.load_gather(w2_v, [rows, tv])
                d1 = xv - w1
                d2 = xv - w2
                acc1[u] = acc1[u] + d1 * d1
                acc2[u] = acc2[u] + d2 * d2
            return tuple(acc1) + tuple(acc2)

        accs = lax.fori_loop(0, D // NACC, body, (zero,) * (2 * NACC),
                             unroll=2)
        acc1 = list(accs[:NACC])
        acc2 = list(accs[NACC:])
        step = NACC
        while step > 1:
            step //= 2
            for u in range(step):
                acc1[u] = acc1[u] + acc1[u + step]
                acc2[u] = acc2[u] + acc2[u + step]
        d1 = acc1[0]
        d2 = acc2[0]
        take2 = (d2 < d1) | ((d2 == d1) & (i2v < i1v))
        win = jnp.where(take2, i2v, i1v)
        gx = plsc.load_gather(grid_v, [win, zeros_i])
        gy = plsc.load_gather(grid_v, [win, ones_i])
        plsc.store_scatter(out_v, [rows, zeros_i], gx)
        plsc.store_scatter(out_v, [rows, ones_i], gy)

    pltpu.sync_copy(out_v, out_hbm.at[pl.ds(base, BPW)])


@functools.cache
def _sc_refine():
    return pl.kernel(
        _sc_refine_body,
        mesh=plsc.VectorSubcoreMesh(
            core_axis_name="c", subcore_axis_name="s",
            num_cores=NC, num_subcores=NS,
        ),
        compiler_params=pltpu.CompilerParams(
            needs_layout_passes=False, use_tc_tiling_on_sc=False,
        ),
        out_type=jax.ShapeDtypeStruct((B, 2), jnp.int32),
        scratch_types=[
            pltpu.VMEM((BPW,), jnp.int32),
            pltpu.VMEM((BPW,), jnp.int32),
            pltpu.VMEM((BPW, D), jnp.float32),
            pltpu.VMEM((BPW, D), jnp.float32),
            pltpu.VMEM((BPW, D), jnp.float32),
            pltpu.VMEM((K, 2), jnp.int32),
            pltpu.VMEM((BPW, 2), jnp.int32),
            pltpu.SemaphoreType.DMA,
            pltpu.SemaphoreType.DMA,
        ],
    )


def kernel(inputs, weights, grid):
    i1, i2 = _tc_top2(weights, inputs)
    return _sc_refine()(inputs, weights, grid, i1, i2)
